# baseline (speedup 1.0000x reference)
import functools

import jax
import jax.numpy as jnp
from jax.experimental import pallas as pl
from jax.experimental.pallas import tpu as pltpu

_K = 9
_NLEVELS = 5
_BIG_I = 2**30
_INF = float("inf")


def _locate(z, v, ids):
    return jnp.min(jnp.where(z == v, ids, _BIG_I), axis=1, keepdims=True)


def _body(emb_ref, bank_ref, outv_ref, outi_ref, score_ref, best_ref, z_ref,
          v_ref, *, C, nM, nQ):
    q = pl.program_id(0)
    m = pl.program_id(1)

    x = emb_ref[...]
    y = bank_ref[...]
    x2 = jnp.sum(x * x, axis=1, keepdims=True)
    y2 = jnp.sum(y * y, axis=1)[None, :]
    xy = jax.lax.dot_general(x, y, (((1,), (1,)), ((), ())),
                             preferred_element_type=jnp.float32)
    d2 = x2 + y2 - 2.0 * xy

    ids = jax.lax.broadcasted_iota(jnp.int32, d2.shape, 1) + m * C

    @pl.when(m == 0)
    def _first_chunk():
        z = d2
        newv, newi = [], []
        for _ in range(_K):
            v = jnp.min(z, axis=1, keepdims=True)
            i = _locate(z, v, ids)
            newv.append(v)
            newi.append(i)
            z = jnp.where(ids == i, _INF, z)
        outv_ref[...] = jnp.concatenate(newv, axis=1)
        outi_ref[...] = jnp.concatenate(newi, axis=1)

    @pl.when(m > 0)
    def _merge_chunk():
        z_ref[...] = d2
        v_ref[...] = jnp.min(d2, axis=1, keepdims=True)

        def _level(j):
            v = v_ref[...]
            go = jnp.any(v <= outv_ref[:, _K - 1:_K])

            @pl.when(go)
            def _insert():
                z = z_ref[...]
                vv = v_ref[...]
                for _ in range(2):
                    cv = outv_ref[...]
                    ci = outi_ref[...]
                    i = _locate(z, vv, ids)
                    lt = (vv < cv) | ((vv == cv) & (i < ci))
                    lti = lt.astype(jnp.int32)
                    ltm1 = jnp.concatenate(
                        [jnp.zeros_like(lti[:, 0:1]), lti[:, :_K - 1]], axis=1)
                    f = (lti - ltm1) == 1
                    cvs = jnp.concatenate([cv[:, 0:1], cv[:, :_K - 1]], axis=1)
                    cis = jnp.concatenate([ci[:, 0:1], ci[:, :_K - 1]], axis=1)
                    outv_ref[...] = jnp.where(lt, jnp.where(f, vv, cvs), cv)
                    outi_ref[...] = jnp.where(lt, jnp.where(f, i, cis), ci)
                    z = jnp.where(ids == i, _INF, z)
                    vv = jnp.min(z, axis=1, keepdims=True)
                z_ref[...] = z
                v_ref[...] = vv
                if j + 1 < _NLEVELS:
                    _level(j + 1)

        _level(0)

    @pl.when(m == nM - 1)
    def _finalize_tile():
        vfin = outv_ref[...]
        col0 = vfin[:, 0:1]
        tmax = jnp.max(col0, axis=0, keepdims=True)
        riota = jax.lax.broadcasted_iota(jnp.int32, col0.shape, 0)
        ridx = jnp.min(jnp.where(col0 == tmax, riota, _BIG_I),
                       axis=0, keepdims=True)
        row9 = jnp.sum(jnp.where(riota == ridx, vfin, 0.0),
                       axis=0, keepdims=True)

        outv_ref[...] = jnp.sqrt(jnp.maximum(vfin, 1e-12))

        @pl.when(q == 0)
        def _():
            best_ref[0:1, 0:_K] = row9
            best_ref[1:2, 0:1] = tmax

        @pl.when(q > 0)
        def _():
            prev = best_ref[1:2, 0:1]
            take = tmax > prev
            best_ref[0:1, 0:_K] = jnp.where(take, row9, best_ref[0:1, 0:_K])
            best_ref[1:2, 0:1] = jnp.where(take, tmax, prev)

        @pl.when(q == nQ - 1)
        def _():
            s = jnp.sqrt(jnp.maximum(best_ref[0:1, 0:_K], 1e-12))
            e = jnp.exp(s - s[:, _K - 1:_K])
            w = 1.0 - e[:, 0:1] / jnp.sum(e, axis=1, keepdims=True)
            score_ref[...] = w * s[:, 0:1]


def kernel(embedding, memory_bank):
    Q, D = embedding.shape
    M = memory_bank.shape[0]
    QT = 448 if Q % 448 == 0 else Q
    C = 2048 if M % 2048 == 0 else M
    nQ, nM = Q // QT, M // C

    outv, outi, score = pl.pallas_call(
        functools.partial(_body, C=C, nM=nM, nQ=nQ),
        grid=(nQ, nM),
        in_specs=[
            pl.BlockSpec((QT, D), lambda q, m: (q, 0)),
            pl.BlockSpec((C, D), lambda q, m: (m, 0)),
        ],
        out_specs=[
            pl.BlockSpec((QT, _K), lambda q, m: (q, 0)),
            pl.BlockSpec((QT, _K), lambda q, m: (q, 0)),
            pl.BlockSpec((1, 1), lambda q, m: (0, 0)),
        ],
        out_shape=[
            jax.ShapeDtypeStruct((Q, _K), jnp.float32),
            jax.ShapeDtypeStruct((Q, _K), jnp.int32),
            jax.ShapeDtypeStruct((1, 1), jnp.float32),
        ],
        scratch_shapes=[pltpu.VMEM((8, 128), jnp.float32),
                        pltpu.VMEM((QT, C), jnp.float32),
                        pltpu.VMEM((QT, 1), jnp.float32)],
        compiler_params=pltpu.CompilerParams(
            dimension_semantics=("arbitrary", "arbitrary")),
    )(embedding, memory_bank)
    return outv, outi, score[0, 0]


# speedup vs baseline: 1.0963x; 1.0963x over previous
import functools

import jax
import jax.numpy as jnp
from jax.experimental import pallas as pl
from jax.experimental.pallas import tpu as pltpu

_K = 9
_NLEVELS = 5
_BIG_I = 2**30
_INF = float("inf")


def _locate(z, v, ids):
    return jnp.min(jnp.where(z == v, ids, _BIG_I), axis=1, keepdims=True)


def _body(emb_ref, bank_ref, outv_ref, outi_ref, score_ref, best_ref, zh_ref,
          zx_ref, pid_ref, v_ref, *, C, nM, nQ):
    q = pl.program_id(0)
    m = pl.program_id(1)

    x = emb_ref[...]
    y = bank_ref[...]
    x2 = jnp.sum(x * x, axis=1, keepdims=True)
    y2 = jnp.sum(y * y, axis=1)[None, :]
    xy = jax.lax.dot_general(x, y, (((1,), (1,)), ((), ())),
                             preferred_element_type=jnp.float32)
    d2 = x2 + y2 - 2.0 * xy

    H = C // 2
    za = d2[:, :H]
    zb = d2[:, H:]
    side = zb < za
    ida = jax.lax.broadcasted_iota(jnp.int32, za.shape, 1) + m * C
    zh_ref[...] = jnp.minimum(za, zb)
    zx_ref[...] = jnp.maximum(za, zb)
    pid_ref[...] = jnp.where(side, ida + H, ida)
    v_ref[...] = jnp.min(zh_ref[...], axis=1, keepdims=True)

    @pl.when(m == 0)
    def _init():
        outv_ref[...] = jnp.full(outv_ref.shape, _INF, jnp.float32)
        outi_ref[...] = jnp.zeros(outi_ref.shape, jnp.int32)

    def _level(j):
        v = v_ref[...]
        go = jnp.any(v <= outv_ref[:, _K - 1:_K])

        @pl.when(go)
        def _insert():
            zh = zh_ref[...]
            zx = zx_ref[...]
            pid = pid_ref[...]
            cv = outv_ref[...]
            ci = outi_ref[...]
            i = jnp.min(jnp.where(zh == v, pid, _BIG_I), axis=1, keepdims=True)
            lt = (v < cv) | ((v == cv) & (i < ci))
            lti = lt.astype(jnp.int32)
            ltm1 = jnp.concatenate(
                [jnp.zeros_like(lti[:, 0:1]), lti[:, :_K - 1]], axis=1)
            f = (lti - ltm1) == 1
            cvs = jnp.concatenate([cv[:, 0:1], cv[:, :_K - 1]], axis=1)
            cis = jnp.concatenate([ci[:, 0:1], ci[:, :_K - 1]], axis=1)
            outv_ref[...] = jnp.where(lt, jnp.where(f, v, cvs), cv)
            outi_ref[...] = jnp.where(lt, jnp.where(f, i, cis), ci)
            hit = pid == i
            zhn = jnp.where(hit, zx, zh)
            zh_ref[...] = zhn
            zx_ref[...] = jnp.where(hit, _INF, zx)
            pid_ref[...] = jnp.where(hit, pid ^ H, pid)
            v_ref[...] = jnp.min(zhn, axis=1, keepdims=True)
            if j + 1 < _K:
                _level(j + 1)

    _level(0)

    @pl.when(m == nM - 1)
    def _finalize_tile():
        vfin = outv_ref[...]
        col0 = vfin[:, 0:1]
        tmax = jnp.max(col0, axis=0, keepdims=True)
        riota = jax.lax.broadcasted_iota(jnp.int32, col0.shape, 0)
        ridx = jnp.min(jnp.where(col0 == tmax, riota, _BIG_I),
                       axis=0, keepdims=True)
        row9 = jnp.sum(jnp.where(riota == ridx, vfin, 0.0),
                       axis=0, keepdims=True)

        outv_ref[...] = jnp.sqrt(jnp.maximum(vfin, 1e-12))

        @pl.when(q == 0)
        def _():
            best_ref[0:1, 0:_K] = row9
            best_ref[1:2, 0:1] = tmax

        @pl.when(q > 0)
        def _():
            prev = best_ref[1:2, 0:1]
            take = tmax > prev
            best_ref[0:1, 0:_K] = jnp.where(take, row9, best_ref[0:1, 0:_K])
            best_ref[1:2, 0:1] = jnp.where(take, tmax, prev)

        @pl.when(q == nQ - 1)
        def _():
            s = jnp.sqrt(jnp.maximum(best_ref[0:1, 0:_K], 1e-12))
            e = jnp.exp(s - s[:, _K - 1:_K])
            w = 1.0 - e[:, 0:1] / jnp.sum(e, axis=1, keepdims=True)
            score_ref[...] = w * s[:, 0:1]


def kernel(embedding, memory_bank):
    Q, D = embedding.shape
    M = memory_bank.shape[0]
    QT = 448 if Q % 448 == 0 else Q
    C = 2048 if M % 2048 == 0 else M
    nQ, nM = Q // QT, M // C

    outv, outi, score = pl.pallas_call(
        functools.partial(_body, C=C, nM=nM, nQ=nQ),
        grid=(nQ, nM),
        in_specs=[
            pl.BlockSpec((QT, D), lambda q, m: (q, 0)),
            pl.BlockSpec((C, D), lambda q, m: (m, 0)),
        ],
        out_specs=[
            pl.BlockSpec((QT, _K), lambda q, m: (q, 0)),
            pl.BlockSpec((QT, _K), lambda q, m: (q, 0)),
            pl.BlockSpec((1, 1), lambda q, m: (0, 0)),
        ],
        out_shape=[
            jax.ShapeDtypeStruct((Q, _K), jnp.float32),
            jax.ShapeDtypeStruct((Q, _K), jnp.int32),
            jax.ShapeDtypeStruct((1, 1), jnp.float32),
        ],
        scratch_shapes=[pltpu.VMEM((8, 128), jnp.float32),
                        pltpu.VMEM((QT, C // 2), jnp.float32),
                        pltpu.VMEM((QT, C // 2), jnp.float32),
                        pltpu.VMEM((QT, C // 2), jnp.int32),
                        pltpu.VMEM((QT, 1), jnp.float32)],
        compiler_params=pltpu.CompilerParams(
            dimension_semantics=("arbitrary", "arbitrary")),
    )(embedding, memory_bank)
    return outv, outi, score[0, 0]


# speedup vs baseline: 1.1457x; 1.0450x over previous
import functools

import jax
import jax.numpy as jnp
from jax.experimental import pallas as pl
from jax.experimental.pallas import tpu as pltpu

_K = 9
_NLEVELS = 5
_BIG_I = 2**30
_INF = float("inf")


def _locate(z, v, ids):
    return jnp.min(jnp.where(z == v, ids, _BIG_I), axis=1, keepdims=True)


def _body(emb_ref, bank_ref, outv_ref, outi_ref, score_ref, best_ref, zh_ref,
          zx_ref, pid_ref, v_ref, *, C, nM, nQ):
    q = pl.program_id(0)
    m = pl.program_id(1)

    x = emb_ref[...]
    y = bank_ref[...]
    x2 = jnp.sum(x * x, axis=1, keepdims=True)
    y2 = jnp.sum(y * y, axis=1)[None, :]
    xy = jax.lax.dot_general(x, y, (((1,), (1,)), ((), ())),
                             preferred_element_type=jnp.float32)
    d2 = x2 + y2 - 2.0 * xy

    H = C // 2
    za = d2[:, :H]
    zb = d2[:, H:]
    side = zb < za
    ida = jax.lax.broadcasted_iota(jnp.int32, za.shape, 1) + m * C
    zh_ref[...] = jnp.minimum(za, zb)
    zx_ref[...] = jnp.maximum(za, zb)
    pid_ref[...] = jnp.where(side, ida + H, ida)
    v_ref[...] = jnp.min(zh_ref[...], axis=1, keepdims=True)

    @pl.when(m == 0)
    def _init():
        outv_ref[...] = jnp.full(outv_ref.shape, _INF, jnp.float32)
        outi_ref[...] = jnp.zeros(outi_ref.shape, jnp.int32)

    def _level(j):
        v = v_ref[...]
        go = jnp.any(v <= outv_ref[:, _K - 1:_K])

        @pl.when(go)
        def _insert():
            zh = zh_ref[...]
            zx = zx_ref[...]
            pid = pid_ref[...]
            cv = outv_ref[...]
            ci = outi_ref[...]
            i = jnp.min(jnp.where(zh == v, pid, _BIG_I), axis=1, keepdims=True)
            lt = (v < cv) | ((v == cv) & (i < ci))
            lti = lt.astype(jnp.int32)
            ltm1 = jnp.concatenate(
                [jnp.zeros_like(lti[:, 0:1]), lti[:, :_K - 1]], axis=1)
            f = (lti - ltm1) == 1
            cvs = jnp.concatenate([cv[:, 0:1], cv[:, :_K - 1]], axis=1)
            cis = jnp.concatenate([ci[:, 0:1], ci[:, :_K - 1]], axis=1)
            outv_ref[...] = jnp.where(lt, jnp.where(f, v, cvs), cv)
            outi_ref[...] = jnp.where(lt, jnp.where(f, i, cis), ci)
            hit = pid == i
            zhn = jnp.where(hit, zx, zh)
            zh_ref[...] = zhn
            zx_ref[...] = jnp.where(hit, _INF, zx)
            pid_ref[...] = jnp.where(hit, pid ^ H, pid)
            v_ref[...] = jnp.min(zhn, axis=1, keepdims=True)
            if j + 1 < _K:
                _level(j + 1)

    _level(0)

    @pl.when(m == nM - 1)
    def _finalize_tile():
        vfin = outv_ref[...]
        col0 = vfin[:, 0:1]
        tmax = jnp.max(col0, axis=0, keepdims=True)
        riota = jax.lax.broadcasted_iota(jnp.int32, col0.shape, 0)
        ridx = jnp.min(jnp.where(col0 == tmax, riota, _BIG_I),
                       axis=0, keepdims=True)
        row9 = jnp.sum(jnp.where(riota == ridx, vfin, 0.0),
                       axis=0, keepdims=True)

        outv_ref[...] = jnp.sqrt(jnp.maximum(vfin, 1e-12))

        @pl.when(q == 0)
        def _():
            best_ref[0:1, 0:_K] = row9
            best_ref[1:2, 0:1] = tmax

        @pl.when(q > 0)
        def _():
            prev = best_ref[1:2, 0:1]
            take = tmax > prev
            best_ref[0:1, 0:_K] = jnp.where(take, row9, best_ref[0:1, 0:_K])
            best_ref[1:2, 0:1] = jnp.where(take, tmax, prev)

        @pl.when(q == nQ - 1)
        def _():
            s = jnp.sqrt(jnp.maximum(best_ref[0:1, 0:_K], 1e-12))
            e = jnp.exp(s - s[:, _K - 1:_K])
            w = 1.0 - e[:, 0:1] / jnp.sum(e, axis=1, keepdims=True)
            score_ref[...] = w * s[:, 0:1]


def kernel(embedding, memory_bank):
    Q, D = embedding.shape
    M = memory_bank.shape[0]
    QT = 448 if Q % 448 == 0 else Q
    C = 4096 if M % 4096 == 0 else M
    nQ, nM = Q // QT, M // C

    outv, outi, score = pl.pallas_call(
        functools.partial(_body, C=C, nM=nM, nQ=nQ),
        grid=(nQ, nM),
        in_specs=[
            pl.BlockSpec((QT, D), lambda q, m: (q, 0)),
            pl.BlockSpec((C, D), lambda q, m: (m, 0)),
        ],
        out_specs=[
            pl.BlockSpec((QT, _K), lambda q, m: (q, 0)),
            pl.BlockSpec((QT, _K), lambda q, m: (q, 0)),
            pl.BlockSpec((1, 1), lambda q, m: (0, 0)),
        ],
        out_shape=[
            jax.ShapeDtypeStruct((Q, _K), jnp.float32),
            jax.ShapeDtypeStruct((Q, _K), jnp.int32),
            jax.ShapeDtypeStruct((1, 1), jnp.float32),
        ],
        scratch_shapes=[pltpu.VMEM((8, 128), jnp.float32),
                        pltpu.VMEM((QT, C // 2), jnp.float32),
                        pltpu.VMEM((QT, C // 2), jnp.float32),
                        pltpu.VMEM((QT, C // 2), jnp.int32),
                        pltpu.VMEM((QT, 1), jnp.float32)],
        compiler_params=pltpu.CompilerParams(
            dimension_semantics=("arbitrary", "arbitrary")),
    )(embedding, memory_bank)
    return outv, outi, score[0, 0]
